# Initial kernel scaffold; baseline (speedup 1.0000x reference)
import jax
import jax.numpy as jnp
from jax.experimental import pallas as pl


def kernel(activation):
    raise NotImplementedError("write your pallas kernel here")


# speedup vs baseline: 4.4259x; 4.4259x over previous
import jax
import jax.numpy as jnp
from jax.experimental import pallas as pl
from jax.experimental.pallas import tpu as pltpu

_CB = 16


def _agg(n, b, transpose):
    if transpose:
        j = jax.lax.broadcasted_iota(jnp.int32, (n // b, n), 0)
        i = jax.lax.broadcasted_iota(jnp.int32, (n // b, n), 1)
    else:
        i = jax.lax.broadcasted_iota(jnp.int32, (n, n // b), 0)
        j = jax.lax.broadcasted_iota(jnp.int32, (n, n // b), 1)
    return (i // b == j).astype(jnp.float32)


def _body(x_ref, o_ref):
    cc = pl.program_id(1)
    g = cc // 3

    @pl.when(g == 0)
    def _():
        x = x_ref[...]
        o_ref[...] = x * (x >= 0).astype(x.dtype)

    @pl.when(g == 3)
    def _():
        o_ref[...] = x_ref[...]

    def block_group(b):
        H = x_ref.shape[2]
        R = _agg(H, b, False)
        RT = _agg(H, b, True)

        def body(c, carry):
            xc = x_ref[0, c]
            colsum = jax.lax.dot_general(
                xc, R, (((1,), (0,)), ((), ())),
                precision=jax.lax.Precision.HIGHEST)
            s = jax.lax.dot_general(
                RT, colsum, (((1,), (0,)), ((), ())),
                precision=jax.lax.Precision.HIGHEST)
            m = (s >= 0).astype(jnp.float32)
            e1 = jax.lax.dot_general(R, m, (((1,), (0,)), ((), ())))
            e = jax.lax.dot_general(e1, RT, (((1,), (0,)), ((), ())))
            o_ref[0, c] = xc * e
            return carry

        jax.lax.fori_loop(0, _CB, body, 0)

    @pl.when(g == 1)
    def _():
        block_group(2)

    @pl.when(g == 2)
    def _():
        block_group(4)


def kernel(activation):
    B, C, H, W = activation.shape
    return pl.pallas_call(
        _body,
        grid=(B, C // _CB),
        in_specs=[pl.BlockSpec((1, _CB, H, W), lambda b, c: (b, c, 0, 0))],
        out_specs=pl.BlockSpec((1, _CB, H, W), lambda b, c: (b, c, 0, 0)),
        out_shape=jax.ShapeDtypeStruct(activation.shape, activation.dtype),
        compiler_params=pltpu.CompilerParams(
            dimension_semantics=("parallel", "parallel")),
    )(activation)


# speedup vs baseline: 9.7764x; 2.2089x over previous
import jax
import jax.numpy as jnp
from jax.experimental import pallas as pl
from jax.experimental.pallas import tpu as pltpu

_CB = 16


def _roll(x, shift, axis):
    return pltpu.roll(x, shift % x.shape[axis], axis)


def _pair_sum_bcast(x, axis, b):
    idx = jax.lax.broadcasted_iota(jnp.int32, x.shape, axis)
    if b == 2:
        s = x + _roll(x, -1, axis)
        return jnp.where(idx % 2 == 0, s, _roll(s, 1, axis))
    s1 = x + _roll(x, -1, axis)
    s = s1 + _roll(s1, -2, axis)
    t = jnp.where(idx % 2 == 0, s, _roll(s, 1, axis))
    return jnp.where(idx % 4 < 2, t, _roll(t, 2, axis))


def _body(x_ref, o_ref):
    cc = pl.program_id(1)
    g = cc // 3

    @pl.when(g == 0)
    def _():
        x = x_ref[...]
        o_ref[...] = x * (x >= 0).astype(x.dtype)

    @pl.when(g == 3)
    def _():
        o_ref[...] = x_ref[...]

    def block_group(b):
        c, h, w = x_ref.shape[1], x_ref.shape[2], x_ref.shape[3]
        x = x_ref[...].reshape(c * h, w)
        s = _pair_sum_bcast(x, 1, b)
        s = _pair_sum_bcast(s, 0, b)
        o_ref[...] = (x * (s >= 0).astype(x.dtype)).reshape(1, c, h, w)

    @pl.when(g == 1)
    def _():
        block_group(2)

    @pl.when(g == 2)
    def _():
        block_group(4)


def kernel(activation):
    B, C, H, W = activation.shape
    return pl.pallas_call(
        _body,
        grid=(B, C // _CB),
        in_specs=[pl.BlockSpec((1, _CB, H, W), lambda b, c: (b, c, 0, 0))],
        out_specs=pl.BlockSpec((1, _CB, H, W), lambda b, c: (b, c, 0, 0)),
        out_shape=jax.ShapeDtypeStruct(activation.shape, activation.dtype),
        compiler_params=pltpu.CompilerParams(
            dimension_semantics=("parallel", "parallel")),
    )(activation)


# speedup vs baseline: 12.2022x; 1.2481x over previous
import jax
import jax.numpy as jnp
from jax.experimental import pallas as pl
from jax.experimental.pallas import tpu as pltpu

_CB = 16


def _roll(x, shift, axis):
    return pltpu.roll(x, shift % x.shape[axis], axis)


def _pair_sum_bcast(x, axis, b):
    idx = jax.lax.broadcasted_iota(jnp.int32, x.shape, axis)
    if b == 2:
        s = x + _roll(x, -1, axis)
        return jnp.where(idx % 2 == 0, s, _roll(s, 1, axis))
    s1 = x + _roll(x, -1, axis)
    s = s1 + _roll(s1, -2, axis)
    t = jnp.where(idx % 2 == 0, s, _roll(s, 1, axis))
    return jnp.where(idx % 4 < 2, t, _roll(t, 2, axis))


def _body(x_ref, o_ref):
    cc = pl.program_id(1)
    g = cc // 3

    @pl.when(g == 0)
    def _():
        x = x_ref[...]
        o_ref[...] = x * (x >= 0).astype(x.dtype)

    @pl.when(g == 3)
    def _():
        o_ref[...] = x_ref[...]

    def block_group(b, lanes_on_mxu):
        c, h, w = x_ref.shape[1], x_ref.shape[2], x_ref.shape[3]
        x = x_ref[...].reshape(c * h, w)
        if lanes_on_mxu:
            i = jax.lax.broadcasted_iota(jnp.int32, (w, w), 0)
            j = jax.lax.broadcasted_iota(jnp.int32, (w, w), 1)
            e = (i // b == j // b).astype(jnp.float32)
            s = jax.lax.dot_general(x, e, (((1,), (0,)), ((), ())),
                                    precision=jax.lax.Precision.HIGHEST)
        else:
            s = _pair_sum_bcast(x, 1, b)
        s = _pair_sum_bcast(s, 0, b)
        o_ref[...] = (x * (s >= 0).astype(x.dtype)).reshape(1, c, h, w)

    @pl.when(g == 1)
    def _():
        block_group(2, lanes_on_mxu=False)

    @pl.when(g == 2)
    def _():
        block_group(4, lanes_on_mxu=True)


def kernel(activation):
    B, C, H, W = activation.shape
    return pl.pallas_call(
        _body,
        grid=(B, C // _CB),
        in_specs=[pl.BlockSpec((1, _CB, H, W), lambda b, c: (b, c, 0, 0))],
        out_specs=pl.BlockSpec((1, _CB, H, W), lambda b, c: (b, c, 0, 0)),
        out_shape=jax.ShapeDtypeStruct(activation.shape, activation.dtype),
        compiler_params=pltpu.CompilerParams(
            dimension_semantics=("parallel", "parallel")),
    )(activation)


# speedup vs baseline: 12.2826x; 1.0066x over previous
import jax
import jax.numpy as jnp
from jax.experimental import pallas as pl
from jax.experimental.pallas import tpu as pltpu

_CB = 16


def _roll(x, shift, axis):
    return pltpu.roll(x, shift % x.shape[axis], axis)


def _pair_sum_bcast(x, axis, b):
    idx = jax.lax.broadcasted_iota(jnp.int32, x.shape, axis)
    if b == 2:
        s = x + _roll(x, -1, axis)
        return jnp.where(idx % 2 == 0, s, _roll(s, 1, axis))
    s1 = x + _roll(x, -1, axis)
    s = s1 + _roll(s1, -2, axis)
    t = jnp.where(idx % 2 == 0, s, _roll(s, 1, axis))
    return jnp.where(idx % 4 < 2, t, _roll(t, 2, axis))


def _body(x_ref, o_ref):
    cc = pl.program_id(1)
    g = cc // 3

    @pl.when(g == 0)
    def _():
        x = x_ref[...]
        o_ref[...] = x * (x >= 0).astype(x.dtype)

    @pl.when(g == 3)
    def _():
        o_ref[...] = x_ref[...]

    def block_group(b, lanes_on_mxu):
        c, h, w = x_ref.shape[1], x_ref.shape[2], x_ref.shape[3]
        x = x_ref[...].reshape(c * h, w)
        if lanes_on_mxu:
            i = jax.lax.broadcasted_iota(jnp.int32, (w, w), 0)
            j = jax.lax.broadcasted_iota(jnp.int32, (w, w), 1)
            e = (i // b == j // b).astype(jnp.float32)
            s = jax.lax.dot_general(x, e, (((1,), (0,)), ((), ())),
                                    precision=jax.lax.Precision.HIGHEST)
        else:
            s = _pair_sum_bcast(x, 1, b)
        s = _pair_sum_bcast(s, 0, b)
        o_ref[...] = (x * (s >= 0).astype(x.dtype)).reshape(1, c, h, w)

    @pl.when(g == 1)
    def _():
        block_group(2, lanes_on_mxu=True)

    @pl.when(g == 2)
    def _():
        block_group(4, lanes_on_mxu=True)


def kernel(activation):
    B, C, H, W = activation.shape
    return pl.pallas_call(
        _body,
        grid=(B, C // _CB),
        in_specs=[pl.BlockSpec((1, _CB, H, W), lambda b, c: (b, c, 0, 0))],
        out_specs=pl.BlockSpec((1, _CB, H, W), lambda b, c: (b, c, 0, 0)),
        out_shape=jax.ShapeDtypeStruct(activation.shape, activation.dtype),
        compiler_params=pltpu.CompilerParams(
            dimension_semantics=("parallel", "parallel")),
    )(activation)


# speedup vs baseline: 16.0510x; 1.3068x over previous
import jax
import jax.numpy as jnp
from jax.experimental import pallas as pl
from jax.experimental.pallas import tpu as pltpu

_CB = 16


def _roll(x, shift, axis):
    return pltpu.roll(x, shift % x.shape[axis], axis)


def _pair_sum_bcast(x, axis, b):
    idx = jax.lax.broadcasted_iota(jnp.int32, x.shape, axis)
    if b == 2:
        s = x + _roll(x, -1, axis)
        return jnp.where(idx % 2 == 0, s, _roll(s, 1, axis))
    s1 = x + _roll(x, -1, axis)
    s = s1 + _roll(s1, -2, axis)
    t = jnp.where(idx % 2 == 0, s, _roll(s, 1, axis))
    return jnp.where(idx % 4 < 2, t, _roll(t, 2, axis))


def _body(x_ref, o_ref):
    cc = pl.program_id(1)
    g = cc // 3

    @pl.when(g == 0)
    def _():
        x = x_ref[...]
        o_ref[...] = x * (x >= 0).astype(x.dtype)

    @pl.when(g == 3)
    def _():
        o_ref[...] = x_ref[...]

    def block_group(b, lanes_on_mxu):
        c, h, w = x_ref.shape[1], x_ref.shape[2], x_ref.shape[3]
        x = x_ref[...].reshape(c * h, w)
        if lanes_on_mxu:
            i = jax.lax.broadcasted_iota(jnp.int32, (w, w), 0)
            j = jax.lax.broadcasted_iota(jnp.int32, (w, w), 1)
            e = (i // b == j // b).astype(jnp.bfloat16)
            hi = x.astype(jnp.bfloat16)
            lo = (x - hi.astype(jnp.float32)).astype(jnp.bfloat16)
            f32 = jnp.float32
            s = (jax.lax.dot_general(hi, e, (((1,), (0,)), ((), ())),
                                     preferred_element_type=f32)
                 + jax.lax.dot_general(lo, e, (((1,), (0,)), ((), ())),
                                       preferred_element_type=f32))
        else:
            s = _pair_sum_bcast(x, 1, b)
        s = _pair_sum_bcast(s, 0, b)
        o_ref[...] = (x * (s >= 0).astype(x.dtype)).reshape(1, c, h, w)

    @pl.when(g == 1)
    def _():
        block_group(2, lanes_on_mxu=True)

    @pl.when(g == 2)
    def _():
        block_group(4, lanes_on_mxu=True)


def kernel(activation):
    B, C, H, W = activation.shape
    return pl.pallas_call(
        _body,
        grid=(B, C // _CB),
        in_specs=[pl.BlockSpec((1, _CB, H, W), lambda b, c: (b, c, 0, 0))],
        out_specs=pl.BlockSpec((1, _CB, H, W), lambda b, c: (b, c, 0, 0)),
        out_shape=jax.ShapeDtypeStruct(activation.shape, activation.dtype),
        compiler_params=pltpu.CompilerParams(
            dimension_semantics=("parallel", "parallel")),
    )(activation)


# speedup vs baseline: 16.5707x; 1.0324x over previous
import jax
import jax.numpy as jnp
from jax.experimental import pallas as pl
from jax.experimental.pallas import tpu as pltpu

_CB = 24


def _roll(x, shift, axis):
    return pltpu.roll(x, shift % x.shape[axis], axis)


def _pair_sum_bcast(x, axis, b):
    idx = jax.lax.broadcasted_iota(jnp.int32, x.shape, axis)
    if b == 2:
        s = x + _roll(x, -1, axis)
        return jnp.where(idx % 2 == 0, s, _roll(s, 1, axis))
    s1 = x + _roll(x, -1, axis)
    s = s1 + _roll(s1, -2, axis)
    t = jnp.where(idx % 2 == 0, s, _roll(s, 1, axis))
    return jnp.where(idx % 4 < 2, t, _roll(t, 2, axis))


def _body(x_ref, o_ref):
    cc = pl.program_id(1)
    g = cc // (48 // _CB)

    @pl.when(g == 0)
    def _():
        x = x_ref[...]
        o_ref[...] = x * (x >= 0).astype(x.dtype)

    @pl.when(g == 3)
    def _():
        o_ref[...] = x_ref[...]

    def block_group(b, lanes_on_mxu):
        c, h, w = x_ref.shape[1], x_ref.shape[2], x_ref.shape[3]
        x = x_ref[...].reshape(c * h, w)
        if lanes_on_mxu:
            i = jax.lax.broadcasted_iota(jnp.int32, (w, w), 0)
            j = jax.lax.broadcasted_iota(jnp.int32, (w, w), 1)
            e = (i // b == j // b).astype(jnp.bfloat16)
            hi = x.astype(jnp.bfloat16)
            lo = (x - hi.astype(jnp.float32)).astype(jnp.bfloat16)
            f32 = jnp.float32
            s = (jax.lax.dot_general(hi, e, (((1,), (0,)), ((), ())),
                                     preferred_element_type=f32)
                 + jax.lax.dot_general(lo, e, (((1,), (0,)), ((), ())),
                                       preferred_element_type=f32))
        else:
            s = _pair_sum_bcast(x, 1, b)
        s = _pair_sum_bcast(s, 0, b)
        o_ref[...] = (x * (s >= 0).astype(x.dtype)).reshape(1, c, h, w)

    @pl.when(g == 1)
    def _():
        block_group(2, lanes_on_mxu=True)

    @pl.when(g == 2)
    def _():
        block_group(4, lanes_on_mxu=True)


def kernel(activation):
    B, C, H, W = activation.shape
    return pl.pallas_call(
        _body,
        grid=(B, C // _CB),
        in_specs=[pl.BlockSpec((1, _CB, H, W), lambda b, c: (b, c, 0, 0))],
        out_specs=pl.BlockSpec((1, _CB, H, W), lambda b, c: (b, c, 0, 0)),
        out_shape=jax.ShapeDtypeStruct(activation.shape, activation.dtype),
        compiler_params=pltpu.CompilerParams(
            dimension_semantics=("parallel", "parallel")),
    )(activation)


# speedup vs baseline: 17.6856x; 1.0673x over previous
import jax
import jax.numpy as jnp
from jax.experimental import pallas as pl
from jax.experimental.pallas import tpu as pltpu

_CB = 48


def _roll(x, shift, axis):
    return pltpu.roll(x, shift % x.shape[axis], axis)


def _pair_sum_bcast(x, axis, b):
    idx = jax.lax.broadcasted_iota(jnp.int32, x.shape, axis)
    if b == 2:
        s = x + _roll(x, -1, axis)
        return jnp.where(idx % 2 == 0, s, _roll(s, 1, axis))
    s1 = x + _roll(x, -1, axis)
    s = s1 + _roll(s1, -2, axis)
    t = jnp.where(idx % 2 == 0, s, _roll(s, 1, axis))
    return jnp.where(idx % 4 < 2, t, _roll(t, 2, axis))


def _body(x_ref, o_ref):
    cc = pl.program_id(1)
    g = cc // (48 // _CB)

    @pl.when(g == 0)
    def _():
        x = x_ref[...]
        o_ref[...] = x * (x >= 0).astype(x.dtype)

    @pl.when(g == 3)
    def _():
        o_ref[...] = x_ref[...]

    def block_group(b, lanes_on_mxu):
        c, h, w = x_ref.shape[1], x_ref.shape[2], x_ref.shape[3]
        x = x_ref[...].reshape(c * h, w)
        if lanes_on_mxu:
            i = jax.lax.broadcasted_iota(jnp.int32, (w, w), 0)
            j = jax.lax.broadcasted_iota(jnp.int32, (w, w), 1)
            e = (i // b == j // b).astype(jnp.bfloat16)
            hi = x.astype(jnp.bfloat16)
            lo = (x - hi.astype(jnp.float32)).astype(jnp.bfloat16)
            f32 = jnp.float32
            s = (jax.lax.dot_general(hi, e, (((1,), (0,)), ((), ())),
                                     preferred_element_type=f32)
                 + jax.lax.dot_general(lo, e, (((1,), (0,)), ((), ())),
                                       preferred_element_type=f32))
        else:
            s = _pair_sum_bcast(x, 1, b)
        s = _pair_sum_bcast(s, 0, b)
        o_ref[...] = (x * (s >= 0).astype(x.dtype)).reshape(1, c, h, w)

    @pl.when(g == 1)
    def _():
        block_group(2, lanes_on_mxu=True)

    @pl.when(g == 2)
    def _():
        block_group(4, lanes_on_mxu=True)


def kernel(activation):
    B, C, H, W = activation.shape
    return pl.pallas_call(
        _body,
        grid=(B, C // _CB),
        in_specs=[pl.BlockSpec((1, _CB, H, W), lambda b, c: (b, c, 0, 0))],
        out_specs=pl.BlockSpec((1, _CB, H, W), lambda b, c: (b, c, 0, 0)),
        out_shape=jax.ShapeDtypeStruct(activation.shape, activation.dtype),
        compiler_params=pltpu.CompilerParams(
            dimension_semantics=("parallel", "parallel")),
    )(activation)
